# baseline (speedup 1.0000x reference)
import jax
import jax.numpy as jnp
from jax.experimental import pallas as pl

_HIDDEN = 2048
_NUM_EXPERTS = 16
_BLOCK = 1024


def _router_block(x_ref, wT_ref, scores_ref, ew_ref, idx_ref):
    x = x_ref[...]
    wT = wT_ref[...]
    logits = jnp.dot(x, wT, preferred_element_type=jnp.float32)
    lmax = jnp.max(logits, axis=1, keepdims=True)
    e = jnp.exp(logits - lmax)
    scores = e / jnp.sum(e, axis=1, keepdims=True)
    scores_ref[...] = scores

    iota = jax.lax.broadcasted_iota(jnp.int32, scores.shape, 1)
    m1 = jnp.max(scores, axis=1, keepdims=True)
    i1 = jnp.min(jnp.where(scores == m1, iota, _NUM_EXPERTS), axis=1,
                 keepdims=True)
    masked = jnp.where(iota == i1, -1.0, scores)
    m2 = jnp.max(masked, axis=1, keepdims=True)
    i2 = jnp.min(jnp.where(masked == m2, iota, _NUM_EXPERTS), axis=1,
                 keepdims=True)

    norm = m1 + m2
    ew_ref[...] = jnp.concatenate([m1, m2], axis=1) / norm
    idx_ref[...] = jnp.concatenate([i1, i2], axis=1)


def kernel(x, W):
    n = x.shape[0]
    wT = W.T
    grid = (n // _BLOCK,)
    scores, ew, idx = pl.pallas_call(
        _router_block,
        grid=grid,
        in_specs=[
            pl.BlockSpec((_BLOCK, _HIDDEN), lambda i: (i, 0)),
            pl.BlockSpec((_HIDDEN, _NUM_EXPERTS), lambda i: (0, 0)),
        ],
        out_specs=[
            pl.BlockSpec((_BLOCK, _NUM_EXPERTS), lambda i: (i, 0)),
            pl.BlockSpec((_BLOCK, 2), lambda i: (i, 0)),
            pl.BlockSpec((_BLOCK, 2), lambda i: (i, 0)),
        ],
        out_shape=[
            jax.ShapeDtypeStruct((n, _NUM_EXPERTS), jnp.float32),
            jax.ShapeDtypeStruct((n, 2), jnp.float32),
            jax.ShapeDtypeStruct((n, 2), jnp.int32),
        ],
    )(x, wT)
    return (scores, ew, idx)


# speedup vs baseline: 1.0355x; 1.0355x over previous
import jax
import jax.numpy as jnp
from jax.experimental import pallas as pl

_HIDDEN = 2048
_NUM_EXPERTS = 16
_BLOCK = 1024


def _router_block(x_ref, wT_ref, scores_ref, ew_ref, idx_ref):
    x = x_ref[...]
    wT = wT_ref[...]
    logits = jnp.dot(x, wT, preferred_element_type=jnp.float32)
    lmax = jnp.max(logits, axis=1, keepdims=True)
    e = jnp.exp(logits - lmax)
    scores = e * (1.0 / jnp.sum(e, axis=1, keepdims=True))
    scores_ref[...] = scores

    iota = jax.lax.broadcasted_iota(jnp.int32, scores.shape, 1)
    bits = jax.lax.bitcast_convert_type(scores, jnp.int32)
    packed = jnp.bitwise_or(jnp.bitwise_and(bits, -16), 15 - iota)
    p1 = jnp.max(packed, axis=1, keepdims=True)
    p2 = jnp.max(jnp.where(packed == p1, jnp.int32(-2147483647 - 1), packed),
                 axis=1, keepdims=True)
    i1 = 15 - jnp.bitwise_and(p1, 15)
    i2 = 15 - jnp.bitwise_and(p2, 15)
    v1 = jax.lax.bitcast_convert_type(jnp.bitwise_and(p1, -16), jnp.float32)
    v2 = jax.lax.bitcast_convert_type(jnp.bitwise_and(p2, -16), jnp.float32)

    inv_norm = 1.0 / (v1 + v2)
    ew_ref[...] = jnp.concatenate([v1, v2], axis=1) * inv_norm
    idx_ref[...] = jnp.concatenate([i1, i2], axis=1)


def kernel(x, W):
    n = x.shape[0]
    wT = W.T
    grid = (n // _BLOCK,)
    scores, ew, idx = pl.pallas_call(
        _router_block,
        grid=grid,
        in_specs=[
            pl.BlockSpec((_BLOCK, _HIDDEN), lambda i: (i, 0)),
            pl.BlockSpec((_HIDDEN, _NUM_EXPERTS), lambda i: (0, 0)),
        ],
        out_specs=[
            pl.BlockSpec((_BLOCK, _NUM_EXPERTS), lambda i: (i, 0)),
            pl.BlockSpec((_BLOCK, 2), lambda i: (i, 0)),
            pl.BlockSpec((_BLOCK, 2), lambda i: (i, 0)),
        ],
        out_shape=[
            jax.ShapeDtypeStruct((n, _NUM_EXPERTS), jnp.float32),
            jax.ShapeDtypeStruct((n, 2), jnp.float32),
            jax.ShapeDtypeStruct((n, 2), jnp.int32),
        ],
    )(x, wT)
    return (scores, ew, idx)


# speedup vs baseline: 1.0690x; 1.0324x over previous
import jax
import jax.numpy as jnp
from jax.experimental import pallas as pl

_HIDDEN = 2048
_NUM_EXPERTS = 16
_BLOCK = 2048


def _router_block(x_ref, wT_ref, scores_ref, ew_ref, idx_ref):
    x = x_ref[...]
    wT = wT_ref[...]
    logits = jnp.dot(x, wT, preferred_element_type=jnp.float32)
    lmax = jnp.max(logits, axis=1, keepdims=True)
    e = jnp.exp(logits - lmax)
    scores = e * (1.0 / jnp.sum(e, axis=1, keepdims=True))
    scores_ref[...] = scores

    iota = jax.lax.broadcasted_iota(jnp.int32, scores.shape, 1)
    bits = jax.lax.bitcast_convert_type(scores, jnp.int32)
    packed = jnp.bitwise_or(jnp.bitwise_and(bits, -16), 15 - iota)
    p1 = jnp.max(packed, axis=1, keepdims=True)
    p2 = jnp.max(jnp.where(packed == p1, jnp.int32(-2147483647 - 1), packed),
                 axis=1, keepdims=True)
    i1 = 15 - jnp.bitwise_and(p1, 15)
    i2 = 15 - jnp.bitwise_and(p2, 15)
    v1 = jax.lax.bitcast_convert_type(jnp.bitwise_and(p1, -16), jnp.float32)
    v2 = jax.lax.bitcast_convert_type(jnp.bitwise_and(p2, -16), jnp.float32)

    inv_norm = 1.0 / (v1 + v2)
    ew_ref[...] = jnp.concatenate([v1, v2], axis=1) * inv_norm
    idx_ref[...] = jnp.concatenate([i1, i2], axis=1)


def kernel(x, W):
    n = x.shape[0]
    wT = W.T
    grid = (n // _BLOCK,)
    scores, ew, idx = pl.pallas_call(
        _router_block,
        grid=grid,
        in_specs=[
            pl.BlockSpec((_BLOCK, _HIDDEN), lambda i: (i, 0)),
            pl.BlockSpec((_HIDDEN, _NUM_EXPERTS), lambda i: (0, 0)),
        ],
        out_specs=[
            pl.BlockSpec((_BLOCK, _NUM_EXPERTS), lambda i: (i, 0)),
            pl.BlockSpec((_BLOCK, 2), lambda i: (i, 0)),
            pl.BlockSpec((_BLOCK, 2), lambda i: (i, 0)),
        ],
        out_shape=[
            jax.ShapeDtypeStruct((n, _NUM_EXPERTS), jnp.float32),
            jax.ShapeDtypeStruct((n, 2), jnp.float32),
            jax.ShapeDtypeStruct((n, 2), jnp.int32),
        ],
    )(x, wT)
    return (scores, ew, idx)
